# baseline (speedup 1.0000x reference)
import functools

import jax
import jax.numpy as jnp
from jax import lax
from jax.experimental import pallas as pl
from jax.experimental.pallas import tpu as pltpu
from jax.experimental.pallas import tpu_sc as plsc

N_PAD = 102400
LANE = 128
NB = N_PAD // LANE
BS = 160
NC, NS = 2, 16
N_GROUPS = 10
SITES_PER_W = N_PAD // N_GROUPS
SC_LANE = 16


CHUNK = 5120
CHUNKS_PER_W = SITES_PER_W // CHUNK
UNROLL = 8


N_SITES = 100000
PT_W = 3200
PT_TAIL = N_SITES - 31 * PT_W
PT_CHUNK = 800


def _passthrough(wid, pairs, bufs, sem):
    col0 = wid * PT_W

    def run(cnt):
        ins = [
            pltpu.async_copy(src.at[pl.ds(soff + col0, cnt)],
                             buf.at[pl.ds(0, cnt)], sem)
            for (src, soff, _, _), buf in zip(pairs, bufs)
        ]
        for c in ins:
            c.wait()
        outs = [
            pltpu.async_copy(buf.at[pl.ds(0, cnt)],
                             dst.at[pl.ds(doff + col0, cnt)], sem)
            for (_, _, dst, doff), buf in zip(pairs, bufs)
        ]
        for c in outs:
            c.wait()

    @pl.when(wid < 31)
    def _():
        run(PT_W)

    @pl.when(wid == 31)
    def _():
        run(PT_TAIL)


def _restride_in_body(qp_hbm, dp_hbm, at_hbm, qout_hbm, dout_hbm, atout_hbm,
                      *scratch):
    bufs, (sem,) = scratch[:-1], scratch[-1:]
    cid = lax.axis_index("c")
    sid = lax.axis_index("s")
    wid = sid * NC + cid
    pairs = [(qp_hbm, j * N_SITES, qout_hbm, j * N_PAD) for j in range(9)]
    pairs += [(dp_hbm, j * N_SITES, dout_hbm, j * N_PAD) for j in range(3)]
    pairs += [(at_hbm, 0, atout_hbm, 0)]
    _passthrough(wid, pairs, bufs, sem)


@functools.cache
def _restride_in():
    return pl.kernel(
        _restride_in_body,
        out_type=[jax.ShapeDtypeStruct((9 * N_PAD,), jnp.float32),
                  jax.ShapeDtypeStruct((3 * N_PAD,), jnp.float32),
                  jax.ShapeDtypeStruct((N_PAD,), jnp.int32)],
        mesh=plsc.VectorSubcoreMesh(core_axis_name="c", subcore_axis_name="s"),
        scratch_types=(
            [pltpu.VMEM((PT_W,), jnp.float32) for _ in range(12)]
            + [pltpu.VMEM((PT_W,), jnp.int32), pltpu.SemaphoreType.DMA]
        ),
        compiler_params=pltpu.CompilerParams(use_tc_tiling_on_sc=False,
                                             needs_layout_passes=False),
    )


def _gather_diff_body(coords_hbm, z_hbm, x_hbm, y_hbm, out_hbm,
                      table, ib0, ib1, ob0, ob1, tsem, s0, s1, o0, o1):
    cid = lax.axis_index("c")
    sid = lax.axis_index("s")
    wid = sid * NC + cid
    comp = jnp.minimum(wid // N_GROUPS, 2)
    slot = wid % N_GROUPS
    nk = 3 * CHUNKS_PER_W

    @pl.when(wid < 3 * N_GROUPS)
    def _():
        tcopy = pltpu.async_copy(
            coords_hbm.at[pl.ds(comp * N_SITES, N_SITES)],
            table.at[pl.ds(0, N_SITES)], tsem)
        srcs = (z_hbm, x_hbm, y_hbm)
        ibufs, isems = (ib0, ib1), (s0, s1)
        obufs, osems = (ob0, ob1), (o0, o1)
        base0 = slot * SITES_PER_W

        def in_off(k):
            v, c = divmod(k, CHUNKS_PER_W)
            return base0 + c * CHUNK

        def start_in(k):
            v = k // CHUNKS_PER_W
            return pltpu.async_copy(
                srcs[v].at[pl.ds(in_off(k), CHUNK)], ibufs[k % 2], isems[k % 2])

        icopies = {0: start_in(0), 1: start_in(1)}
        ocopies = {}
        tcopy.wait()
        for k in range(nk):
            ib, ob = ibufs[k % 2], obufs[k % 2]
            icopies[k].wait()
            if k >= 2:
                ocopies[k - 2].wait()
            cbase = in_off(k)

            def body(j, carry, ib=ib, ob=ob, cbase=cbase):
                for u in range(UNROLL):
                    g = j * (SC_LANE * UNROLL) + u * SC_LANE
                    ii = ib[pl.ds(g, SC_LANE)]
                    ii = jnp.minimum(jnp.maximum(ii, 0), N_PAD - 1)
                    vals = plsc.load_gather(table, [ii])
                    s = table[pl.ds(cbase + g, SC_LANE)]
                    ob[pl.ds(g, SC_LANE)] = vals - s
                return carry

            lax.fori_loop(0, CHUNK // (SC_LANE * UNROLL), body, 0)
            if k + 2 < nk:
                icopies[k + 2] = start_in(k + 2)
            v = k // CHUNKS_PER_W
            dst = (3 * v + comp) * N_PAD + in_off(k)
            ocopies[k] = pltpu.async_copy(
                ob, out_hbm.at[pl.ds(dst, CHUNK)], osems[k % 2])
        ocopies[nk - 2].wait()
        ocopies[nk - 1].wait()


@functools.cache
def _gather_diff():
    return pl.kernel(
        _gather_diff_body,
        out_type=jax.ShapeDtypeStruct((9 * N_PAD,), jnp.float32),
        mesh=plsc.VectorSubcoreMesh(core_axis_name="c", subcore_axis_name="s"),
        scratch_types=[
            pltpu.VMEM((N_PAD,), jnp.float32),
            pltpu.VMEM((CHUNK,), jnp.int32),
            pltpu.VMEM((CHUNK,), jnp.int32),
            pltpu.VMEM((CHUNK,), jnp.float32),
            pltpu.VMEM((CHUNK,), jnp.float32),
            pltpu.SemaphoreType.DMA,
            pltpu.SemaphoreType.DMA,
            pltpu.SemaphoreType.DMA,
            pltpu.SemaphoreType.DMA,
            pltpu.SemaphoreType.DMA,
        ],
        compiler_params=pltpu.CompilerParams(use_tc_tiling_on_sc=False,
                                             needs_layout_passes=False),
    )


def _unpad_body(dt_hbm, qt_hbm, d_hbm, q_hbm, *scratch):
    bufs, (sem,) = scratch[:-1], scratch[-1:]
    cid = lax.axis_index("c")
    sid = lax.axis_index("s")
    wid = sid * NC + cid
    pairs = [(dt_hbm, j * N_PAD, d_hbm, j * N_SITES) for j in range(3)]
    pairs += [(qt_hbm, j * N_PAD, q_hbm, j * N_SITES) for j in range(9)]
    _passthrough(wid, pairs, bufs, sem)


@functools.cache
def _unpad_out():
    return pl.kernel(
        _unpad_body,
        out_type=[jax.ShapeDtypeStruct((3 * N_SITES,), jnp.float32),
                  jax.ShapeDtypeStruct((9 * N_SITES,), jnp.float32)],
        mesh=plsc.VectorSubcoreMesh(core_axis_name="c", subcore_axis_name="s"),
        scratch_types=(
            [pltpu.VMEM((PT_W,), jnp.float32) for _ in range(12)]
            + [pltpu.SemaphoreType.DMA]
        ),
        compiler_params=pltpu.CompilerParams(use_tc_tiling_on_sc=False,
                                             needs_layout_passes=False),
    )


def _normalize3(vx, vy, vz):
    ss = vx * vx + vy * vy + vz * vz
    r = jnp.where(ss > 0, lax.rsqrt(ss), jnp.float32(0.0))
    return vx * r, vy * r, vz * r


def _math_body(diff_ref, at_ref, dip_ref, quad_ref, d_ref, q_ref):
    at = at_ref[0]
    dzx, dzy, dzz = diff_ref[0], diff_ref[1], diff_ref[2]
    dxx, dxy, dxz = diff_ref[3], diff_ref[4], diff_ref[5]
    dyx, dyy, dyz = diff_ref[6], diff_ref[7], diff_ref[8]

    zx, zy, zz = _normalize3(dzx, dzy, dzz)
    fzo = (at == 4) | (at == 5)
    xnx, xny, xnz = _normalize3(dxx, dxy, dxz)
    one = jnp.float32(1.0)
    zero = jnp.float32(0.0)
    xx = jnp.where(fzo, one - zx, xnx)
    xy = jnp.where(fzo, zx, xny)
    xz = jnp.where(fzo, zero, xnz)
    fb = at == 1
    zx, zy, zz = _normalize3(
        jnp.where(fb, zx + xx, zx),
        jnp.where(fb, zy + xy, zy),
        jnp.where(fb, zz + xz, zz),
    )
    ynx, yny, ynz = _normalize3(dyx, dyy, dyz)
    fzb = at == 2
    sx, sy, sz = _normalize3(xx + ynx, xy + yny, xz + ynz)
    xx = jnp.where(fzb, sx, xx)
    xy = jnp.where(fzb, sy, xy)
    xz = jnp.where(fzb, sz, xz)
    f3 = at == 3
    tx, ty, tz = _normalize3(zx + xx + ynx, zy + xy + yny, zz + xz + ynz)
    zx = jnp.where(f3, tx, zx)
    zy = jnp.where(f3, ty, zy)
    zz = jnp.where(f3, tz, zz)
    dot = zx * xx + zy * xy + zz * xz
    xx, xy, xz = _normalize3(xx - zx * dot, xy - zy * dot, xz - zz * dot)
    yx = zy * xz - zz * xy
    yy = zz * xx - zx * xz
    yz = zx * xy - zy * xx
    fna = at == 5
    zx = jnp.where(fna, zero, zx)
    zy = jnp.where(fna, zero, zy)
    zz = jnp.where(fna, one, zz)
    xx = jnp.where(fna, one, xx)
    xy = jnp.where(fna, zero, xy)
    xz = jnp.where(fna, zero, xz)
    yx = jnp.where(fna, zero, yx)
    yy = jnp.where(fna, one, yy)
    yz = jnp.where(fna, zero, yz)

    R = ((xx, xy, xz), (yx, yy, yz), (zx, zy, zz))
    dip = (dip_ref[0], dip_ref[1], dip_ref[2])
    for k in range(3):
        d_ref[k] = dip[0] * R[0][k] + dip[1] * R[1][k] + dip[2] * R[2][k]
    Q = tuple(tuple(quad_ref[3 * j + k] for k in range(3)) for j in range(3))
    M = tuple(
        tuple(Q[j][0] * R[0][l] + Q[j][1] * R[1][l] + Q[j][2] * R[2][l]
              for l in range(3))
        for j in range(3)
    )
    for i in range(3):
        for l in range(3):
            q_ref[3 * i + l] = (R[0][i] * M[0][l] + R[1][i] * M[1][l]
                                + R[2][i] * M[2][l])


def _spec(rows):
    return pl.BlockSpec((rows, BS, LANE), lambda i: (0, i, 0))


_math_call = pl.pallas_call(
    _math_body,
    grid=(NB // BS,),
    in_specs=[_spec(9), _spec(1), _spec(3), _spec(9)],
    out_specs=[_spec(3), _spec(9)],
    out_shape=[
        jax.ShapeDtypeStruct((3, NB, LANE), jnp.float32),
        jax.ShapeDtypeStruct((9, NB, LANE), jnp.float32),
    ],
)


def kernel(coords, z_atoms, x_atoms, y_atoms, axis_types, dipoles, quadrupoles):
    n = coords.shape[0]
    f32 = jnp.float32
    ct = coords.T.astype(f32).reshape(-1)
    qp = quadrupoles.astype(f32).transpose(1, 2, 0).reshape(-1)
    dp = dipoles.T.astype(f32).reshape(-1)

    quadp, dipp, atp = _restride_in()(qp, dp, axis_types.astype(jnp.int32))
    diffs = _gather_diff()(
        ct, z_atoms.astype(jnp.int32), x_atoms.astype(jnp.int32),
        y_atoms.astype(jnp.int32))
    d_t, q_t = _math_call(
        diffs.reshape(9, NB, LANE),
        atp.reshape(1, NB, LANE),
        dipp.reshape(3, NB, LANE),
        quadp.reshape(9, NB, LANE),
    )
    d_pl, q_pl = _unpad_out()(d_t.reshape(-1), q_t.reshape(-1))
    d = d_pl.reshape(3, n).T
    q = q_pl.reshape(3, 3, n).transpose(2, 0, 1)
    return d, q


# speedup vs baseline: 1.0244x; 1.0244x over previous
import functools

import jax
import jax.numpy as jnp
from jax import lax
from jax.experimental import pallas as pl
from jax.experimental.pallas import tpu as pltpu
from jax.experimental.pallas import tpu_sc as plsc

N_PAD = 102400
LANE = 128
NB = N_PAD // LANE
BS = 200
NC, NS = 2, 16
N_GROUPS = 10
SITES_PER_W = N_PAD // N_GROUPS
SC_LANE = 16


CHUNK = 5120
CHUNKS_PER_W = SITES_PER_W // CHUNK
UNROLL = 4


N_SITES = 100000
PT_W = 3200
PT_TAIL = N_SITES - 31 * PT_W
PT_CHUNK = 800


def _passthrough(wid, pairs, bufs, sem):
    col0 = wid * PT_W

    def run(cnt):
        ins = [
            pltpu.async_copy(src.at[pl.ds(soff + col0, cnt)],
                             buf.at[pl.ds(0, cnt)], sem)
            for (src, soff, _, _), buf in zip(pairs, bufs)
        ]
        for c in ins:
            c.wait()
        outs = [
            pltpu.async_copy(buf.at[pl.ds(0, cnt)],
                             dst.at[pl.ds(doff + col0, cnt)], sem)
            for (_, _, dst, doff), buf in zip(pairs, bufs)
        ]
        for c in outs:
            c.wait()

    @pl.when(wid < 31)
    def _():
        run(PT_W)

    @pl.when(wid == 31)
    def _():
        run(PT_TAIL)


def _restride_in_body(qp_hbm, dp_hbm, at_hbm, qout_hbm, dout_hbm, atout_hbm,
                      *scratch):
    bufs, (sem,) = scratch[:-1], scratch[-1:]
    cid = lax.axis_index("c")
    sid = lax.axis_index("s")
    wid = sid * NC + cid
    pairs = [(qp_hbm, j * N_SITES, qout_hbm, j * N_PAD) for j in range(9)]
    pairs += [(dp_hbm, j * N_SITES, dout_hbm, j * N_PAD) for j in range(3)]
    pairs += [(at_hbm, 0, atout_hbm, 0)]
    _passthrough(wid, pairs, bufs, sem)


@functools.cache
def _restride_in():
    return pl.kernel(
        _restride_in_body,
        out_type=[jax.ShapeDtypeStruct((9 * N_PAD,), jnp.float32),
                  jax.ShapeDtypeStruct((3 * N_PAD,), jnp.float32),
                  jax.ShapeDtypeStruct((N_PAD,), jnp.int32)],
        mesh=plsc.VectorSubcoreMesh(core_axis_name="c", subcore_axis_name="s"),
        scratch_types=(
            [pltpu.VMEM((PT_W,), jnp.float32) for _ in range(12)]
            + [pltpu.VMEM((PT_W,), jnp.int32), pltpu.SemaphoreType.DMA]
        ),
        compiler_params=pltpu.CompilerParams(use_tc_tiling_on_sc=False,
                                             needs_layout_passes=False),
    )


def _gather_diff_body(coords_hbm, z_hbm, x_hbm, y_hbm, out_hbm,
                      table, ib0, ib1, ob0, ob1, tsem, s0, s1, o0, o1):
    cid = lax.axis_index("c")
    sid = lax.axis_index("s")
    wid = sid * NC + cid
    comp = jnp.minimum(wid // N_GROUPS, 2)
    slot = wid % N_GROUPS
    nk = 3 * CHUNKS_PER_W

    @pl.when(wid < 3 * N_GROUPS)
    def _():
        tcopy = pltpu.async_copy(
            coords_hbm.at[pl.ds(comp * N_SITES, N_SITES)],
            table.at[pl.ds(0, N_SITES)], tsem)
        srcs = (z_hbm, x_hbm, y_hbm)
        ibufs, isems = (ib0, ib1), (s0, s1)
        obufs, osems = (ob0, ob1), (o0, o1)
        base0 = slot * SITES_PER_W

        def in_off(k):
            v, c = divmod(k, CHUNKS_PER_W)
            return base0 + c * CHUNK

        def start_in(k):
            v = k // CHUNKS_PER_W
            return pltpu.async_copy(
                srcs[v].at[pl.ds(in_off(k), CHUNK)], ibufs[k % 2], isems[k % 2])

        icopies = {0: start_in(0), 1: start_in(1)}
        ocopies = {}
        tcopy.wait()
        for k in range(nk):
            ib, ob = ibufs[k % 2], obufs[k % 2]
            icopies[k].wait()
            if k >= 2:
                ocopies[k - 2].wait()
            cbase = in_off(k)

            def body(j, carry, ib=ib, ob=ob, cbase=cbase):
                for u in range(UNROLL):
                    g = j * (SC_LANE * UNROLL) + u * SC_LANE
                    ii = ib[pl.ds(g, SC_LANE)]
                    ii = jnp.minimum(jnp.maximum(ii, 0), N_PAD - 1)
                    vals = plsc.load_gather(table, [ii])
                    s = table[pl.ds(cbase + g, SC_LANE)]
                    ob[pl.ds(g, SC_LANE)] = vals - s
                return carry

            lax.fori_loop(0, CHUNK // (SC_LANE * UNROLL), body, 0)
            if k + 2 < nk:
                icopies[k + 2] = start_in(k + 2)
            v = k // CHUNKS_PER_W
            dst = (3 * v + comp) * N_PAD + in_off(k)
            ocopies[k] = pltpu.async_copy(
                ob, out_hbm.at[pl.ds(dst, CHUNK)], osems[k % 2])
        ocopies[nk - 2].wait()
        ocopies[nk - 1].wait()


@functools.cache
def _gather_diff():
    return pl.kernel(
        _gather_diff_body,
        out_type=jax.ShapeDtypeStruct((9 * N_PAD,), jnp.float32),
        mesh=plsc.VectorSubcoreMesh(core_axis_name="c", subcore_axis_name="s"),
        scratch_types=[
            pltpu.VMEM((N_PAD,), jnp.float32),
            pltpu.VMEM((CHUNK,), jnp.int32),
            pltpu.VMEM((CHUNK,), jnp.int32),
            pltpu.VMEM((CHUNK,), jnp.float32),
            pltpu.VMEM((CHUNK,), jnp.float32),
            pltpu.SemaphoreType.DMA,
            pltpu.SemaphoreType.DMA,
            pltpu.SemaphoreType.DMA,
            pltpu.SemaphoreType.DMA,
            pltpu.SemaphoreType.DMA,
        ],
        compiler_params=pltpu.CompilerParams(use_tc_tiling_on_sc=False,
                                             needs_layout_passes=False),
    )


def _unpad_body(dt_hbm, qt_hbm, d_hbm, q_hbm, *scratch):
    bufs, (sem,) = scratch[:-1], scratch[-1:]
    cid = lax.axis_index("c")
    sid = lax.axis_index("s")
    wid = sid * NC + cid
    pairs = [(dt_hbm, j * N_PAD, d_hbm, j * N_SITES) for j in range(3)]
    pairs += [(qt_hbm, j * N_PAD, q_hbm, j * N_SITES) for j in range(9)]
    _passthrough(wid, pairs, bufs, sem)


@functools.cache
def _unpad_out():
    return pl.kernel(
        _unpad_body,
        out_type=[jax.ShapeDtypeStruct((3 * N_SITES,), jnp.float32),
                  jax.ShapeDtypeStruct((9 * N_SITES,), jnp.float32)],
        mesh=plsc.VectorSubcoreMesh(core_axis_name="c", subcore_axis_name="s"),
        scratch_types=(
            [pltpu.VMEM((PT_W,), jnp.float32) for _ in range(12)]
            + [pltpu.SemaphoreType.DMA]
        ),
        compiler_params=pltpu.CompilerParams(use_tc_tiling_on_sc=False,
                                             needs_layout_passes=False),
    )


def _normalize3(vx, vy, vz):
    ss = vx * vx + vy * vy + vz * vz
    r = jnp.where(ss > 0, lax.rsqrt(ss), jnp.float32(0.0))
    return vx * r, vy * r, vz * r


def _math_body(diff_ref, at_ref, dip_ref, quad_ref, d_ref, q_ref):
    at = at_ref[0]
    dzx, dzy, dzz = diff_ref[0], diff_ref[1], diff_ref[2]
    dxx, dxy, dxz = diff_ref[3], diff_ref[4], diff_ref[5]
    dyx, dyy, dyz = diff_ref[6], diff_ref[7], diff_ref[8]

    zx, zy, zz = _normalize3(dzx, dzy, dzz)
    fzo = (at == 4) | (at == 5)
    xnx, xny, xnz = _normalize3(dxx, dxy, dxz)
    one = jnp.float32(1.0)
    zero = jnp.float32(0.0)
    xx = jnp.where(fzo, one - zx, xnx)
    xy = jnp.where(fzo, zx, xny)
    xz = jnp.where(fzo, zero, xnz)
    fb = at == 1
    zx, zy, zz = _normalize3(
        jnp.where(fb, zx + xx, zx),
        jnp.where(fb, zy + xy, zy),
        jnp.where(fb, zz + xz, zz),
    )
    ynx, yny, ynz = _normalize3(dyx, dyy, dyz)
    fzb = at == 2
    sx, sy, sz = _normalize3(xx + ynx, xy + yny, xz + ynz)
    xx = jnp.where(fzb, sx, xx)
    xy = jnp.where(fzb, sy, xy)
    xz = jnp.where(fzb, sz, xz)
    f3 = at == 3
    tx, ty, tz = _normalize3(zx + xx + ynx, zy + xy + yny, zz + xz + ynz)
    zx = jnp.where(f3, tx, zx)
    zy = jnp.where(f3, ty, zy)
    zz = jnp.where(f3, tz, zz)
    dot = zx * xx + zy * xy + zz * xz
    xx, xy, xz = _normalize3(xx - zx * dot, xy - zy * dot, xz - zz * dot)
    yx = zy * xz - zz * xy
    yy = zz * xx - zx * xz
    yz = zx * xy - zy * xx
    fna = at == 5
    zx = jnp.where(fna, zero, zx)
    zy = jnp.where(fna, zero, zy)
    zz = jnp.where(fna, one, zz)
    xx = jnp.where(fna, one, xx)
    xy = jnp.where(fna, zero, xy)
    xz = jnp.where(fna, zero, xz)
    yx = jnp.where(fna, zero, yx)
    yy = jnp.where(fna, one, yy)
    yz = jnp.where(fna, zero, yz)

    R = ((xx, xy, xz), (yx, yy, yz), (zx, zy, zz))
    dip = (dip_ref[0], dip_ref[1], dip_ref[2])
    for k in range(3):
        d_ref[k] = dip[0] * R[0][k] + dip[1] * R[1][k] + dip[2] * R[2][k]
    Q = tuple(tuple(quad_ref[3 * j + k] for k in range(3)) for j in range(3))
    M = tuple(
        tuple(Q[j][0] * R[0][l] + Q[j][1] * R[1][l] + Q[j][2] * R[2][l]
              for l in range(3))
        for j in range(3)
    )
    for i in range(3):
        for l in range(3):
            q_ref[3 * i + l] = (R[0][i] * M[0][l] + R[1][i] * M[1][l]
                                + R[2][i] * M[2][l])


def _spec(rows):
    return pl.BlockSpec((rows, BS, LANE), lambda i: (0, i, 0))


_math_call = pl.pallas_call(
    _math_body,
    grid=(NB // BS,),
    in_specs=[_spec(9), _spec(1), _spec(3), _spec(9)],
    out_specs=[_spec(3), _spec(9)],
    out_shape=[
        jax.ShapeDtypeStruct((3, NB, LANE), jnp.float32),
        jax.ShapeDtypeStruct((9, NB, LANE), jnp.float32),
    ],
)


def kernel(coords, z_atoms, x_atoms, y_atoms, axis_types, dipoles, quadrupoles):
    n = coords.shape[0]
    f32 = jnp.float32
    ct = coords.T.astype(f32).reshape(-1)
    qp = quadrupoles.astype(f32).transpose(1, 2, 0).reshape(-1)
    dp = dipoles.T.astype(f32).reshape(-1)

    quadp, dipp, atp = _restride_in()(qp, dp, axis_types.astype(jnp.int32))
    diffs = _gather_diff()(
        ct, z_atoms.astype(jnp.int32), x_atoms.astype(jnp.int32),
        y_atoms.astype(jnp.int32))
    d_t, q_t = _math_call(
        diffs.reshape(9, NB, LANE),
        atp.reshape(1, NB, LANE),
        dipp.reshape(3, NB, LANE),
        quadp.reshape(9, NB, LANE),
    )
    d_pl, q_pl = _unpad_out()(d_t.reshape(-1), q_t.reshape(-1))
    d = d_pl.reshape(3, n).T
    q = q_pl.reshape(3, 3, n).transpose(2, 0, 1)
    return d, q
